# baseline (speedup 1.0000x reference)
import functools

import jax
import jax.numpy as jnp
from jax import lax
from jax.experimental import pallas as pl
from jax.experimental.pallas import tpu as pltpu
from jax.experimental.pallas import tpu_sc as plsc

N = 10000
E = 320000
F = 128
H = 2

NC = 2
NS = 16

G = 80
NG = E // G
GC = 8
NB = 4

NGA = 2000
EA = NGA * G
NGB_C = (NG - NGA) // NC

ROWS_A = 632
ROWS_B = N - 15 * ROWS_A

BN = 1280
BN2 = 2000


def _make_sc_scatter(ngrp, idx_base_fn, src_base_fn, col_fn):
    nch_total = ngrp // GC
    base, extra = nch_total // NS, nch_total % NS
    mesh = plsc.VectorSubcoreMesh(core_axis_name="c", subcore_axis_name="s")

    @functools.partial(
        pl.kernel,
        out_type=jax.ShapeDtypeStruct((NC, N, F), jnp.float32),
        mesh=mesh,
        scratch_types=[
            pltpu.VMEM((GC, G), jnp.int32),
            [pltpu.VMEM((G, F), jnp.float32)] * NB,
            pltpu.VMEM_SHARED((N, F), jnp.float32),
            [pltpu.SemaphoreType.DMA] * NB,
            [pltpu.SemaphoreType.DMA] * NB,
        ],
    )
    def sc_scatter(idx_hbm, sv_hbm, zeros_hbm, out_hbm,
                   idx_buf, ubufs, acc, gsems, ssems):
        c = lax.axis_index("c")
        s = lax.axis_index("s")
        col = col_fn(c)
        r0 = pl.multiple_of(s * ROWS_A, 8)

        m0 = base * s + jnp.minimum(s, extra)
        nch = base + jnp.where(s < extra, 1, 0)

        def src_slice(flat):
            e0 = pl.multiple_of((src_base_fn(c) + m0 * GC + flat) * G, 8)
            return sv_hbm.at[pl.ds(e0, G), pl.ds(col, F)]

        def gissue(flat, b):
            pltpu.async_copy(src_slice(flat), ubufs[b], gsems[b])

        def gwait(flat, b):
            pltpu.make_async_copy(src_slice(flat), ubufs[b], gsems[b]).wait()

        def sissue(b, j):
            pltpu.async_copy(ubufs[b], acc.at[idx_buf.at[j]], ssems[b],
                             add=True)

        def swait(b):
            pltpu.make_async_copy(ubufs[b], acc.at[idx_buf.at[0]],
                                  ssems[b]).wait()

        gissue(0, 0)
        gissue(1, 1)

        @pl.when(s < NS - 1)
        def _():
            pltpu.sync_copy(zeros_hbm, acc.at[pl.ds(r0, ROWS_A), :])

        @pl.when(s == NS - 1)
        def _():
            pltpu.sync_copy(zeros_hbm.at[pl.ds(0, ROWS_B), :],
                            acc.at[pl.ds(15 * ROWS_A, ROWS_B), :])

        plsc.subcore_barrier()

        ng = nch * GC

        def chunk_body(k, carry):
            for j in range(GC):
                b = j % NB
                flat = k * GC + j
                if j == 0:
                    @pl.when(k > 0)
                    def _():
                        swait(2)
                        swait(3)
                    gg = pl.multiple_of(idx_base_fn(c) + (m0 + k) * GC, 8)
                    pltpu.sync_copy(idx_hbm.at[pl.ds(gg, GC), :], idx_buf)
                elif j >= 2:
                    swait((j - 2) % NB)

                @pl.when(flat + 2 < ng)
                def _():
                    gissue(flat + 2, (j + 2) % NB)

                gwait(flat, b)
                sissue(b, j)
            return carry

        lax.fori_loop(0, nch, chunk_body, 0)
        swait(2)
        swait(3)

        plsc.subcore_barrier()

        @pl.when(s < NS - 1)
        def _():
            pltpu.sync_copy(acc.at[pl.ds(r0, ROWS_A), :],
                            out_hbm.at[c, pl.ds(r0, ROWS_A), :])

        @pl.when(s == NS - 1)
        def _():
            pltpu.sync_copy(acc.at[pl.ds(15 * ROWS_A, ROWS_B), :],
                            out_hbm.at[c, pl.ds(15 * ROWS_A, ROWS_B), :])

    return sc_scatter


_sc_scatter_a = _make_sc_scatter(
    NGA,
    idx_base_fn=lambda c: 0,
    src_base_fn=lambda c: 0,
    col_fn=lambda c: pl.multiple_of(c * F, F),
)

_sc_scatter_b = _make_sc_scatter(
    NGB_C,
    idx_base_fn=lambda c: NGA + c * NGB_C,
    src_base_fn=lambda c: c * NGB_C,
    col_fn=lambda c: 0,
)


def _mm_y_body(sv_ref, w_ref, y_ref):
    y_ref[...] = jnp.dot(sv_ref[...], w_ref[...],
                         preferred_element_type=jnp.float32)


def _tc_mm_y(sv, W):
    grid = ((E - EA) // BN,)
    off = EA // BN
    return pl.pallas_call(
        _mm_y_body,
        grid=grid,
        in_specs=[
            pl.BlockSpec((BN, H * F), lambda i: (off + i, 0)),
            pl.BlockSpec((H * F, F), lambda i: (0, 0)),
        ],
        out_specs=pl.BlockSpec((BN, F), lambda i: (i, 0)),
        out_shape=jax.ShapeDtypeStruct((E - EA, F), jnp.float32),
    )(sv, W)


def _part_body(xp_ref, x_ref, w_ref, b_ref, o_ref):
    o_ref[...] = (
        x_ref[...]
        + b_ref[...]
        + jnp.dot(xp_ref[0], w_ref[0], preferred_element_type=jnp.float32)
        + jnp.dot(xp_ref[1], w_ref[1], preferred_element_type=jnp.float32)
    )


def _tc_partial(xp, x_i, W2, b2):
    grid = (N // BN2,)
    return pl.pallas_call(
        _part_body,
        grid=grid,
        in_specs=[
            pl.BlockSpec((H, BN2, F), lambda i: (0, i, 0)),
            pl.BlockSpec((BN2, F), lambda i: (i, 0)),
            pl.BlockSpec((H, F, F), lambda i: (0, 0, 0)),
            pl.BlockSpec((1, F), lambda i: (0, 0)),
        ],
        out_specs=pl.BlockSpec((BN2, F), lambda i: (i, 0)),
        out_shape=jax.ShapeDtypeStruct((N, F), jnp.float32),
    )(xp, x_i, W2, b2)


def _final_body(p_ref, a_ref, o_ref):
    o_ref[...] = p_ref[...] + a_ref[0] + a_ref[1]


def _tc_final(part, accB):
    grid = (N // BN2,)
    return pl.pallas_call(
        _final_body,
        grid=grid,
        in_specs=[
            pl.BlockSpec((BN2, F), lambda i: (i, 0)),
            pl.BlockSpec((H, BN2, F), lambda i: (0, i, 0)),
        ],
        out_specs=pl.BlockSpec((BN2, F), lambda i: (i, 0)),
        out_shape=jax.ShapeDtypeStruct((N, F), jnp.float32),
    )(part, accB)


def kernel(nbrs, x_i, scaled_v, W, b):
    idx2d = nbrs[:, 0].astype(jnp.int32).reshape(NG, G)
    zeros = jnp.zeros((ROWS_A, F), jnp.float32)
    xpA = _sc_scatter_a(idx2d, scaled_v, zeros)
    yB = _tc_mm_y(scaled_v, W)
    accB = _sc_scatter_b(idx2d, yB, zeros)
    part = _tc_partial(xpA, x_i, W.reshape(H, F, F),
                       b.reshape(1, F))
    return _tc_final(part, accB)


# speedup vs baseline: 1.1070x; 1.1070x over previous
import functools

import jax
import jax.numpy as jnp
from jax import lax
from jax.experimental import pallas as pl
from jax.experimental.pallas import tpu as pltpu
from jax.experimental.pallas import tpu_sc as plsc

N = 10000
E = 320000
F = 128
H = 2

NC = 2
NS = 16

G = 128
NG = E // G
GC = 8
NCH_FULL = NG // GC
NG_PAD = (NCH_FULL + 1) * GC

ROWS_A = 632
ROWS_B = N - 15 * ROWS_A


def _make_sc_scatter():
    mesh = plsc.VectorSubcoreMesh(core_axis_name="c", subcore_axis_name="s")

    @functools.partial(
        pl.kernel,
        out_type=jax.ShapeDtypeStruct((NC, N, F), jnp.float32),
        mesh=mesh,
        scratch_types=[
            pltpu.VMEM((GC, G), jnp.int32),
            [pltpu.VMEM((G, F), jnp.float32)] * 2,
            pltpu.VMEM_SHARED((N, F), jnp.float32),
            [pltpu.SemaphoreType.DMA] * 2,
            [pltpu.SemaphoreType.DMA] * 2,
        ],
    )
    def sc_scatter(idx_hbm, sv_hbm, zeros_hbm, out_hbm,
                   idx_buf, ubufs, acc, gsems, ssems):
        c = lax.axis_index("c")
        s = lax.axis_index("s")
        col = pl.multiple_of(c * F, F)
        r0 = pl.multiple_of(s * ROWS_A, 8)

        m0 = jnp.where(s < 8, 20 * s, 160 + 19 * (s - 8))
        nch = jnp.where(s < 8, 20, 19)
        g0 = m0 * GC
        ng = nch * GC

        def src_slice(flat):
            e0 = pl.multiple_of((g0 + flat) * G, 8)
            return sv_hbm.at[pl.ds(e0, G), pl.ds(col, F)]

        def gissue(flat, b):
            pltpu.async_copy(src_slice(flat), ubufs[b], gsems[b])

        def gwait(flat, b):
            pltpu.make_async_copy(src_slice(flat), ubufs[b], gsems[b]).wait()

        def sissue(b, j):
            pltpu.async_copy(ubufs[b], acc.at[idx_buf.at[j]], ssems[b],
                             add=True)

        def swait(b):
            pltpu.make_async_copy(ubufs[b], acc.at[idx_buf.at[0]],
                                  ssems[b]).wait()

        gissue(0, 0)

        @pl.when(s < NS - 1)
        def _():
            pltpu.sync_copy(zeros_hbm, acc.at[pl.ds(r0, ROWS_A), :])

        @pl.when(s == NS - 1)
        def _():
            pltpu.sync_copy(zeros_hbm.at[pl.ds(0, ROWS_B), :],
                            acc.at[pl.ds(15 * ROWS_A, ROWS_B), :])

        plsc.subcore_barrier()

        def chunk_body(k, carry):
            for j in range(GC):
                b = j % 2
                flat = k * GC + j
                if j == 0:
                    @pl.when(k > 0)
                    def _():
                        swait(1)
                    gg = pl.multiple_of((m0 + k) * GC, 8)
                    pltpu.sync_copy(idx_hbm.at[pl.ds(gg, GC), :], idx_buf)
                else:
                    swait(1 - b)

                @pl.when(flat + 1 < ng)
                def _():
                    gissue(flat + 1, 1 - b)

                gwait(flat, b)
                sissue(b, j)
            return carry

        lax.fori_loop(0, nch, chunk_body, 0)
        swait(1)

        @pl.when(s == NS - 1)
        def _():
            gg = NCH_FULL * GC
            pltpu.sync_copy(idx_hbm.at[pl.ds(gg, GC), :], idx_buf)
            for j in range(NG - NCH_FULL * GC):
                e0 = (gg + j) * G
                pltpu.sync_copy(sv_hbm.at[pl.ds(e0, G), pl.ds(col, F)],
                                ubufs[0])
                pltpu.sync_copy(ubufs[0], acc.at[idx_buf.at[j]], add=True)

        plsc.subcore_barrier()

        @pl.when(s < NS - 1)
        def _():
            pltpu.sync_copy(acc.at[pl.ds(r0, ROWS_A), :],
                            out_hbm.at[c, pl.ds(r0, ROWS_A), :])

        @pl.when(s == NS - 1)
        def _():
            pltpu.sync_copy(acc.at[pl.ds(15 * ROWS_A, ROWS_B), :],
                            out_hbm.at[c, pl.ds(15 * ROWS_A, ROWS_B), :])

    return sc_scatter


_sc_scatter = _make_sc_scatter()


def _mm_body(xp_ref, x_ref, w_ref, b_ref, o_ref):
    o_ref[...] = (
        x_ref[...]
        + b_ref[...]
        + jnp.dot(xp_ref[0], w_ref[0], preferred_element_type=jnp.float32)
        + jnp.dot(xp_ref[1], w_ref[1], preferred_element_type=jnp.float32)
    )


def _tc_dense(xp, x_i, W2, b2):
    BN = 2000
    grid = (N // BN,)
    return pl.pallas_call(
        _mm_body,
        grid=grid,
        in_specs=[
            pl.BlockSpec((H, BN, F), lambda i: (0, i, 0)),
            pl.BlockSpec((BN, F), lambda i: (i, 0)),
            pl.BlockSpec((H, F, F), lambda i: (0, 0, 0)),
            pl.BlockSpec((1, F), lambda i: (0, 0)),
        ],
        out_specs=pl.BlockSpec((BN, F), lambda i: (i, 0)),
        out_shape=jax.ShapeDtypeStruct((N, F), jnp.float32),
    )(xp, x_i, W2, b2)


def kernel(nbrs, x_i, scaled_v, W, b):
    idx2d = nbrs[:, 0].astype(jnp.int32).reshape(NG, G)
    idx2d = jnp.pad(idx2d, ((0, NG_PAD - NG), (0, 0)))
    zeros = jnp.zeros((ROWS_A, F), jnp.float32)
    xp = _sc_scatter(idx2d, scaled_v, zeros)
    return _tc_dense(xp, x_i, W.reshape(H, F, F), b.reshape(1, F))


# speedup vs baseline: 1.1464x; 1.0356x over previous
import functools

import jax
import jax.numpy as jnp
from jax import lax
from jax.experimental import pallas as pl
from jax.experimental.pallas import tpu as pltpu
from jax.experimental.pallas import tpu_sc as plsc

N = 10000
E = 320000
F = 128
H = 2

NC = 2
NS = 16

G = 128
NG = E // G
GC = 16
NCH_FULL = NG // GC
NG_PAD = (NCH_FULL + 1) * GC

ROWS_A = 632
ROWS_B = N - 15 * ROWS_A


def _make_sc_scatter():
    mesh = plsc.VectorSubcoreMesh(core_axis_name="c", subcore_axis_name="s")

    @functools.partial(
        pl.kernel,
        out_type=jax.ShapeDtypeStruct((NC, N, F), jnp.float32),
        mesh=mesh,
        scratch_types=[
            pltpu.VMEM((GC, G), jnp.int32),
            [pltpu.VMEM((G, F), jnp.float32)] * 2,
            pltpu.VMEM_SHARED((N, F), jnp.float32),
            [pltpu.SemaphoreType.DMA] * 2,
        ],
    )
    def sc_scatter(idx_hbm, sv_hbm, zeros_hbm, out_hbm,
                   idx_buf, ubufs, acc, gsems):
        c = lax.axis_index("c")
        s = lax.axis_index("s")
        col = pl.multiple_of(c * F, F)
        r0 = pl.multiple_of(s * ROWS_A, 8)

        m0 = jnp.where(s < 12, 10 * s, 120 + 9 * (s - 12))
        nch = jnp.where(s < 12, 10, 9)
        g0 = m0 * GC
        ng = nch * GC

        def src_slice(flat):
            e0 = pl.multiple_of((g0 + flat) * G, 8)
            return sv_hbm.at[pl.ds(e0, G), pl.ds(col, F)]

        def gissue(flat, b):
            pltpu.async_copy(src_slice(flat), ubufs[b], gsems[b])

        def gwait(flat, b):
            pltpu.make_async_copy(src_slice(flat), ubufs[b], gsems[b]).wait()

        gissue(0, 0)
        gissue(1, 1)

        @pl.when(s < NS - 1)
        def _():
            pltpu.sync_copy(zeros_hbm, acc.at[pl.ds(r0, ROWS_A), :])

        @pl.when(s == NS - 1)
        def _():
            pltpu.sync_copy(zeros_hbm.at[pl.ds(0, ROWS_B), :],
                            acc.at[pl.ds(15 * ROWS_A, ROWS_B), :])

        plsc.subcore_barrier()

        def chunk_body(k, carry):
            gg = pl.multiple_of((m0 + k) * GC, 8)
            pltpu.sync_copy(idx_hbm.at[pl.ds(gg, GC), :], idx_buf)
            for j in range(GC):
                b = j % 2
                flat = k * GC + j
                gwait(flat, b)
                pltpu.sync_copy(ubufs[b], acc.at[idx_buf.at[j]], add=True)

                @pl.when(flat + 2 < ng)
                def _():
                    gissue(flat + 2, b)
            return carry

        lax.fori_loop(0, nch, chunk_body, 0)

        @pl.when(s == NS - 1)
        def _():
            gg = NCH_FULL * GC
            pltpu.sync_copy(idx_hbm.at[pl.ds(gg, GC), :], idx_buf)
            for j in range(NG - NCH_FULL * GC):
                e0 = (gg + j) * G
                pltpu.sync_copy(sv_hbm.at[pl.ds(e0, G), pl.ds(col, F)],
                                ubufs[0])
                pltpu.sync_copy(ubufs[0], acc.at[idx_buf.at[j]], add=True)

        plsc.subcore_barrier()

        @pl.when(s < NS - 1)
        def _():
            pltpu.sync_copy(acc.at[pl.ds(r0, ROWS_A), :],
                            out_hbm.at[c, pl.ds(r0, ROWS_A), :])

        @pl.when(s == NS - 1)
        def _():
            pltpu.sync_copy(acc.at[pl.ds(15 * ROWS_A, ROWS_B), :],
                            out_hbm.at[c, pl.ds(15 * ROWS_A, ROWS_B), :])

    return sc_scatter


_sc_scatter = _make_sc_scatter()


def _mm_body(xp_ref, x_ref, w_ref, b_ref, o_ref):
    o_ref[...] = (
        x_ref[...]
        + b_ref[...]
        + jnp.dot(xp_ref[0], w_ref[0], preferred_element_type=jnp.float32)
        + jnp.dot(xp_ref[1], w_ref[1], preferred_element_type=jnp.float32)
    )


def _tc_dense(xp, x_i, W2, b2):
    BN = 2000
    grid = (N // BN,)
    return pl.pallas_call(
        _mm_body,
        grid=grid,
        in_specs=[
            pl.BlockSpec((H, BN, F), lambda i: (0, i, 0)),
            pl.BlockSpec((BN, F), lambda i: (i, 0)),
            pl.BlockSpec((H, F, F), lambda i: (0, 0, 0)),
            pl.BlockSpec((1, F), lambda i: (0, 0)),
        ],
        out_specs=pl.BlockSpec((BN, F), lambda i: (i, 0)),
        out_shape=jax.ShapeDtypeStruct((N, F), jnp.float32),
    )(xp, x_i, W2, b2)


def kernel(nbrs, x_i, scaled_v, W, b):
    idx2d = nbrs[:, 0].astype(jnp.int32).reshape(NG, G)
    idx2d = jnp.pad(idx2d, ((0, NG_PAD - NG), (0, 0)))
    zeros = jnp.zeros((ROWS_A, F), jnp.float32)
    xp = _sc_scatter(idx2d, scaled_v, zeros)
    return _tc_dense(xp, x_i, W.reshape(H, F, F), b.reshape(1, F))
